# Initial kernel scaffold; baseline (speedup 1.0000x reference)
import jax
import jax.numpy as jnp
from jax.experimental import pallas as pl


def kernel(activations, sample_points):
    raise NotImplementedError("write your pallas kernel here")


# speedup vs baseline: 104.9136x; 104.9136x over previous
import functools

import jax
import jax.numpy as jnp
import numpy as np
from jax import lax
from jax.experimental import pallas as pl
from jax.experimental.pallas import tpu as pltpu
from jax.experimental.pallas import tpu_sc as plsc

N_IN = 1048576
N_OUT = 524288

_DELTA = np.float32(1.0) / np.float32(N_IN - 1)
_C = np.float32(N_IN - 1)


_TBL_ROWS = 8192
_TBL_BLK = 512


def _round_bf16_bits(u):
    return (u + jnp.uint32(0x7FFF) + ((u >> 16) & jnp.uint32(1))) & jnp.uint32(
        0xFFFF0000
    )


def _table_body(act_ref, prev_ref, out_ref):
    b = pl.program_id(0)
    a = jnp.clip(act_ref[...], 0.0, 1.0)
    ap = jnp.clip(prev_ref[...], 0.0, 1.0)
    rows = lax.broadcasted_iota(jnp.int32, (_TBL_BLK, 128), 0)
    cols = lax.broadcasted_iota(jnp.int32, (_TBL_BLK, 128), 1)
    i_i = (rows + b * _TBL_BLK) * 128 + cols
    i_f = lax.convert_element_type(i_i, jnp.float32)
    posdiff = i_f * _DELTA - (i_f - jnp.float32(1.0)) * _DELTA
    s = (a - ap) / posdiff
    s = jnp.where(i_f == 0.0, jnp.float32(0.0), s)
    vb = _round_bf16_bits(lax.bitcast_convert_type(a, jnp.uint32))
    sb = _round_bf16_bits(lax.bitcast_convert_type(s, jnp.uint32))
    out_ref[...] = vb | (sb >> 16)


def _build_table(act):
    act2d = act.reshape(_TBL_ROWS, 128)
    prev2d = jnp.concatenate([act[:1], act[:-1]]).reshape(_TBL_ROWS, 128)
    tbl = pl.pallas_call(
        _table_body,
        grid=(_TBL_ROWS // _TBL_BLK,),
        in_specs=[
            pl.BlockSpec((_TBL_BLK, 128), lambda b: (b, 0)),
            pl.BlockSpec((_TBL_BLK, 128), lambda b: (b, 0)),
        ],
        out_specs=pl.BlockSpec((_TBL_BLK, 128), lambda b: (b, 0)),
        out_shape=jax.ShapeDtypeStruct((_TBL_ROWS, 128), jnp.uint32),
    )(act2d, prev2d)
    return tbl.reshape(N_IN)


_NW = 32
_S = (2 * N_OUT) // _NW
_CH = _S // 128
_K = 16


def _searchsorted16(x):
    xi = lax.convert_element_type(x * _C, jnp.int32)
    xf = lax.convert_element_type(xi, jnp.float32)
    one = jnp.float32(1.0)
    zero = jnp.float32(0.0)
    idxf = xf
    idxf = idxf + jnp.where(xf * _DELTA < x, one, zero)
    idxf = idxf + jnp.where((xf + 1.0) * _DELTA < x, one, zero)
    idxf = idxf + jnp.where((xf + 2.0) * _DELTA < x, one, zero)
    return idxf


def _sc_body(tbl_hbm, xe_hbm, xo_hbm, out_hbm, x_v, idx_v, g_v, out_v, sem):
    wid = lax.axis_index("s") * 2 + lax.axis_index("c")
    half = _S // 2
    base = wid * half
    pltpu.sync_copy(xe_hbm.at[pl.ds(base, half)], x_v.at[pl.ds(0, half)])
    pltpu.sync_copy(xo_hbm.at[pl.ds(base, half)], x_v.at[pl.ds(half, half)])

    def idx_body(c, _):
        for u in range(8):
            off = c * 128 + u * 16
            x = jnp.clip(x_v[pl.ds(off, 16)], 0.0, 1.0)
            idx_v[c, pl.ds(u * 16, 16)] = lax.convert_element_type(
                _searchsorted16(x), jnp.int32
            )
        return 0

    lax.fori_loop(0, _CH, idx_body, 0)

    def gather_body(bt, _):
        handles = []
        for u in range(_K):
            c = bt * _K + u
            handles.append(
                pltpu.async_copy(tbl_hbm.at[idx_v.at[c]], g_v.at[c], sem)
            )
        for h in handles:
            h.wait()
        return 0

    lax.fori_loop(0, _CH // _K, gather_body, 0)

    himask = jnp.uint32(0xFFFF0000)

    def interp_body(c, _):
        for u in range(8):
            off = c * 128 + u * 16
            x = jnp.clip(x_v[pl.ds(off, 16)], 0.0, 1.0)
            g = g_v[c, pl.ds(u * 16, 16)]
            idxf = lax.convert_element_type(
                idx_v[c, pl.ds(u * 16, 16)], jnp.float32
            )
            v = lax.bitcast_convert_type(g & himask, jnp.float32)
            s = lax.bitcast_convert_type(g << 16, jnp.float32)
            x_v[pl.ds(off, 16)] = v + s * (x - idxf * _DELTA)
        return 0

    lax.fori_loop(0, _CH, interp_body, 0)

    def mean_body(k, _):
        o = k * 16
        ye = x_v[pl.ds(o, 16)]
        yo = x_v[pl.ds(half + o, 16)]
        out_v[pl.ds(o, 16)] = (ye + yo) * jnp.float32(0.5)
        return 0

    lax.fori_loop(0, half // 16, mean_body, 0)

    pltpu.sync_copy(out_v, out_hbm.at[pl.ds(base, half)])


@functools.partial(jax.jit, static_argnames=())
def _interp_sc(tbl, xe, xo):
    mesh = plsc.VectorSubcoreMesh(core_axis_name="c", subcore_axis_name="s")
    f = pl.kernel(
        _sc_body,
        out_type=jax.ShapeDtypeStruct((N_OUT,), jnp.float32),
        mesh=mesh,
        scratch_types=[
            pltpu.VMEM((_S,), jnp.float32),
            pltpu.VMEM((_CH, 128), jnp.int32),
            pltpu.VMEM((_CH, 128), jnp.uint32),
            pltpu.VMEM((_S // 2,), jnp.float32),
            pltpu.SemaphoreType.DMA,
        ],
    )
    return f(tbl, xe, xo)


def kernel(activations, sample_points):
    tbl = _build_table(activations)
    xe = sample_points[:, 0]
    xo = sample_points[:, 1]
    return _interp_sc(tbl, xe, xo)
